# Initial kernel scaffold; baseline (speedup 1.0000x reference)
import jax
import jax.numpy as jnp
from jax.experimental import pallas as pl


def kernel(x, W_linear, ffm_tables, W1, b1, g1, beta1, W2, b2, g2, beta2, W3, b3, b):
    raise NotImplementedError("write your pallas kernel here")


# speedup vs baseline: 104.1466x; 104.1466x over previous
import numpy as np
import jax
import jax.numpy as jnp
from jax import lax
from jax.experimental import pallas as pl
from jax.experimental.pallas import tpu as pltpu
from jax.experimental.pallas import tpu_sc as plsc

B = 4096
F = 26
VOCAB = 1000
D = 16
P = F * (F - 1) // 2
H1 = 1024
H2 = 512
EPS = 1e-5
SLAB = F * D
FP = 32
SLABP = FP * D

_IU, _JU = np.triu_indices(F, k=1)
_IU = [int(v) for v in _IU]
_JU = [int(v) for v in _JU]

NC, NS = 2, 16
NW = NC * NS
ROWS_W = B // NW
CH = 8
SUB = 4
NCHUNK = ROWS_W // CH


def _sc_gather_body(table_hbm, wlin_hbm, idx_hbm, em_hbm, lin_hbm,
                    idx_v, slabs_v, em_v, lin_v, sem1, sem2):
    wid = lax.axis_index("s") * NC + lax.axis_index("c")
    row0 = wid * ROWS_W

    def chunk_body(c, carry):
        base_b = row0 + c * CH
        for half in range(CH // SUB):
            basei = (base_b + half * SUB) * F
            pltpu.sync_copy(idx_hbm.at[pl.ds(basei, SUB * F)], idx_v)
            cp1 = pltpu.make_async_copy(table_hbm.at[idx_v], slabs_v, sem1)
            cp1.start()
            cp2 = pltpu.make_async_copy(wlin_hbm.at[idx_v], lin_v, sem2)
            cp2.start()
            cp1.wait()

            def row_body(r, carry2):
                sb = r * F
                s = half * SUB + r
                for p in range(P):
                    i = _IU[p]
                    j = _JU[p]
                    a = slabs_v[sb + i, pl.ds(j * D, D)]
                    bv = slabs_v[sb + j, pl.ds(i * D, D)]
                    em_v[s, pl.ds(p * D, D)] = a * bv
                return carry2

            lax.fori_loop(0, SUB, row_body, 0)
            cp2.wait()
            pltpu.sync_copy(lin_v, lin_hbm.at[pl.ds(basei, SUB * F)])
        pltpu.sync_copy(em_v, em_hbm.at[pl.ds(base_b, CH)])
        return carry

    lax.fori_loop(0, NCHUNK, chunk_body, 0)


import functools


@functools.cache
def _sc_gather_fn():
    return pl.kernel(
        _sc_gather_body,
        out_type=(jax.ShapeDtypeStruct((B, P * D), jnp.float32),
                  jax.ShapeDtypeStruct((B * F,), jnp.float32)),
        mesh=plsc.VectorSubcoreMesh(core_axis_name="c", subcore_axis_name="s",
                                    num_cores=NC, num_subcores=NS),
        scratch_types=(pltpu.VMEM((SUB * F,), jnp.int32),
                       pltpu.VMEM((SUB * F, SLABP), jnp.float32),
                       pltpu.VMEM((CH, P * D), jnp.float32),
                       pltpu.VMEM((SUB * F,), jnp.float32),
                       pltpu.SemaphoreType.DMA,
                       pltpu.SemaphoreType.DMA),
    )

BB = 512


def _l1_body(em_ref, w1_ref, h1_ref, s1_ref, q1_ref):
    h = jnp.dot(em_ref[...], w1_ref[...], preferred_element_type=jnp.float32)
    h1_ref[...] = h

    @pl.when(pl.program_id(0) == 0)
    def _init():
        s1_ref[...] = jnp.zeros_like(s1_ref)
        q1_ref[...] = jnp.zeros_like(q1_ref)

    s1_ref[...] += jnp.sum(h, axis=0, keepdims=True)
    q1_ref[...] += jnp.sum(h * h, axis=0, keepdims=True)


_l1 = pl.pallas_call(
    _l1_body,
    grid=(B // BB,),
    in_specs=[pl.BlockSpec((BB, P * D), lambda i: (i, 0)),
              pl.BlockSpec((P * D, H1), lambda i: (0, 0))],
    out_specs=[pl.BlockSpec((BB, H1), lambda i: (i, 0)),
               pl.BlockSpec((1, H1), lambda i: (0, 0)),
               pl.BlockSpec((1, H1), lambda i: (0, 0))],
    out_shape=[jax.ShapeDtypeStruct((B, H1), jnp.float32),
               jax.ShapeDtypeStruct((1, H1), jnp.float32),
               jax.ShapeDtypeStruct((1, H1), jnp.float32)],
)


def _l2_body(h1_ref, s1_ref, q1_ref, g1_ref, bt1_ref, w2_ref,
             h2_ref, s2_ref, q2_ref):
    mu = s1_ref[...] * (1.0 / B)
    var = q1_ref[...] * (1.0 / B) - mu * mu
    scale = g1_ref[...] * lax.rsqrt(var + EPS)
    hn = jnp.maximum(h1_ref[...] * scale + (bt1_ref[...] - mu * scale), 0.0)
    h2 = jnp.dot(hn, w2_ref[...], preferred_element_type=jnp.float32)
    h2_ref[...] = h2

    @pl.when(pl.program_id(0) == 0)
    def _init():
        s2_ref[...] = jnp.zeros_like(s2_ref)
        q2_ref[...] = jnp.zeros_like(q2_ref)

    s2_ref[...] += jnp.sum(h2, axis=0, keepdims=True)
    q2_ref[...] += jnp.sum(h2 * h2, axis=0, keepdims=True)


_l2 = pl.pallas_call(
    _l2_body,
    grid=(B // BB,),
    in_specs=[pl.BlockSpec((BB, H1), lambda i: (i, 0)),
              pl.BlockSpec((1, H1), lambda i: (0, 0)),
              pl.BlockSpec((1, H1), lambda i: (0, 0)),
              pl.BlockSpec((1, H1), lambda i: (0, 0)),
              pl.BlockSpec((1, H1), lambda i: (0, 0)),
              pl.BlockSpec((H1, H2), lambda i: (0, 0))],
    out_specs=[pl.BlockSpec((BB, H2), lambda i: (i, 0)),
               pl.BlockSpec((1, H2), lambda i: (0, 0)),
               pl.BlockSpec((1, H2), lambda i: (0, 0))],
    out_shape=[jax.ShapeDtypeStruct((B, H2), jnp.float32),
               jax.ShapeDtypeStruct((1, H2), jnp.float32),
               jax.ShapeDtypeStruct((1, H2), jnp.float32)],
)


def _l3_body(h2_ref, s2_ref, q2_ref, g2_ref, bt2_ref, w3_ref, lin_ref, c_ref,
             out_ref):
    mu = s2_ref[...] * (1.0 / B)
    var = q2_ref[...] * (1.0 / B) - mu * mu
    scale = g2_ref[...] * lax.rsqrt(var + EPS)
    hn = jnp.maximum(h2_ref[...] * scale + (bt2_ref[...] - mu * scale), 0.0)
    y = jnp.sum(hn * w3_ref[...], axis=1, keepdims=True)
    ylin = jnp.sum(lin_ref[...], axis=1, keepdims=True)
    out_ref[...] = jax.nn.sigmoid(y + ylin + c_ref[...])


_l3 = pl.pallas_call(
    _l3_body,
    grid=(B // BB,),
    in_specs=[pl.BlockSpec((BB, H2), lambda i: (i, 0)),
              pl.BlockSpec((1, H2), lambda i: (0, 0)),
              pl.BlockSpec((1, H2), lambda i: (0, 0)),
              pl.BlockSpec((1, H2), lambda i: (0, 0)),
              pl.BlockSpec((1, H2), lambda i: (0, 0)),
              pl.BlockSpec((1, H2), lambda i: (0, 0)),
              pl.BlockSpec((BB, F), lambda i: (i, 0)),
              pl.BlockSpec((1, 1), lambda i: (0, 0))],
    out_specs=pl.BlockSpec((BB, 1), lambda i: (i, 0)),
    out_shape=jax.ShapeDtypeStruct((B, 1), jnp.float32),
)


def kernel(x, W_linear, ffm_tables, W1, b1, g1, beta1, W2, b2, g2, beta2,
           W3, b3, b):
    table = jnp.pad(ffm_tables.reshape(F * VOCAB, F, D),
                    ((0, 0), (0, FP - F), (0, 0))).reshape(F * VOCAB, SLABP)
    wlin = W_linear.reshape(F * VOCAB)
    idx = (x + jnp.arange(F, dtype=jnp.int32)[None, :] * VOCAB).reshape(-1)
    flat, lin = _sc_gather_fn()(table, wlin, idx)
    h1, s1, q1 = _l1(flat, W1)
    h2, s2, q2 = _l2(h1, s1, q1, g1.reshape(1, H1), beta1.reshape(1, H1), W2)
    c = (b3 + b).reshape(1, 1)
    out = _l3(h2, s2, q2, g2.reshape(1, H2), beta2.reshape(1, H2),
              W3.reshape(1, H2), lin.reshape(B, F), c)
    return out.reshape(B)
